# baseline (speedup 1.0000x reference)
import jax
import jax.numpy as jnp
from jax import lax
from jax.experimental import pallas as pl
from jax.experimental.pallas import tpu as pltpu
from jax.experimental.pallas import tpu_sc as plsc

B = 16384
N_NUM = 13
NUM_OUT = 64
N_CAT = 26
VOCAB = 1000
EMB = 32
OUT_D = NUM_OUT + N_CAT * EMB
TILES = OUT_D // 128
UNITS_PER_BLOCK = TILES * 8 * 4

NC = 2
NS = 16
NW = NC * NS
ROWS_PER_W = B // NW
CHUNK = 32
BLOCKS = CHUNK // 8
UNITS = BLOCKS * UNITS_PER_BLOCK
N_CHUNKS = ROWS_PER_W // CHUNK
LANES = 16
NGATHER = UNITS // 128
NJ = UNITS_PER_BLOCK // LANES
TOTAL_UNITS = (B // 8) * UNITS_PER_BLOCK


def _mm_body(x_ref, w_ref, b_ref, o_ref):
    o_ref[...] = (
        jnp.dot(x_ref[...], w_ref[...], preferred_element_type=jnp.float32)
        + b_ref[...]
    )


def _num_matmul(x, W, b2):
    MB = 2048
    return pl.pallas_call(
        _mm_body,
        grid=(B // MB,),
        in_specs=[
            pl.BlockSpec((MB, N_NUM), lambda i: (i, 0)),
            pl.BlockSpec((N_NUM, NUM_OUT), lambda i: (0, 0)),
            pl.BlockSpec((1, NUM_OUT), lambda i: (0, 0)),
        ],
        out_specs=pl.BlockSpec((MB, NUM_OUT), lambda i: (i, 0)),
        out_shape=jax.ShapeDtypeStruct((B, NUM_OUT), jnp.float32),
    )(x, W, b2)


def _sc_body(num_emb_hbm, cat_hbm, tables_hbm, out_hbm,
             cat_v, num_v, idx0_v, idx1_v, stage0_v, stage1_v,
             gsem0, gsem1, osem0, osem1, ssem):
    cid = lax.axis_index("c")
    sid = lax.axis_index("s")
    wid = sid * NC + cid
    row0 = wid * ROWS_PER_W

    lane = lax.iota(jnp.int32, LANES)
    s_j, f1000_j = [], []
    for j in range(NJ):
        u = lane + (16 * j)
        s = (u >> 2) & 7
        f = ((u >> 5) * 4 + (u & 3)) - 2
        fc = jnp.maximum(f, 0)
        s_j.append(s)
        f1000_j.append(fc * VOCAB)
    fcol_j = [jnp.maximum(((lane + 16 * j) >> 5) * 4 + ((lane + 16 * j) & 3) - 2, 0)
              for j in range(NJ)]

    bufs = [(idx0_v, stage0_v, gsem0, osem0), (idx1_v, stage1_v, gsem1, osem1)]

    def pair_body(t, carry):
        for p in range(2):
            idx_v, stage_v, gsem, osem = bufs[p]
            ci = t * 2 + p
            base = pl.multiple_of(row0 + ci * CHUNK, CHUNK)
            u0 = pl.multiple_of((base // 8) * UNITS_PER_BLOCK, UNITS)
            pltpu.sync_copy(cat_hbm.at[pl.ds(base, CHUNK)], cat_v)
            nstg = pltpu.async_copy(
                num_emb_hbm.at[pl.ds(base, CHUNK)], num_v, ssem
            )
            for blk in range(BLOCKS):
                for j in range(NJ):
                    rows = s_j[j] + (blk * 8)
                    vals = plsc.load_gather(cat_v, [rows, fcol_j[j]])
                    uoff = blk * UNITS_PER_BLOCK + 16 * j
                    idx_v[uoff // 128, pl.ds(uoff % 128, LANES)] = (
                        f1000_j[j] + vals
                    )
            @pl.when(t > 0)
            def _():
                pltpu.make_async_copy(
                    stage_v, out_hbm.at[pl.ds(u0, UNITS)], osem
                ).wait()
            gathers = [
                pltpu.async_copy(
                    tables_hbm.at[idx_v.at[g]],
                    stage_v.at[pl.ds(g * 128, 128)],
                    gsem,
                )
                for g in range(NGATHER)
            ]
            for g in gathers:
                g.wait()
            nstg.wait()
            for blk in range(BLOCKS):
                for s in range(8):
                    r = blk * UNITS_PER_BLOCK + s * 4
                    row = blk * 8 + s
                    for h in range(4):
                        stage_v[r + h // 2, pl.ds((h % 2) * 16, LANES)] = (
                            num_v[row, pl.ds(h * 16, LANES)]
                        )
            pltpu.async_copy(stage_v, out_hbm.at[pl.ds(u0, UNITS)], osem)
        return carry

    lax.fori_loop(0, N_CHUNKS // 2, pair_body, 0)

    for p in range(2):
        idx_v, stage_v, gsem, osem = bufs[p]
        ulast = pl.multiple_of(
            (row0 // 8 + (N_CHUNKS - 2 + p) * BLOCKS) * UNITS_PER_BLOCK, UNITS
        )
        pltpu.make_async_copy(
            stage_v, out_hbm.at[pl.ds(ulast, UNITS)], osem
        ).wait()


_sc_kernel = pl.kernel(
    _sc_body,
    mesh=plsc.VectorSubcoreMesh(core_axis_name="c", subcore_axis_name="s"),
    compiler_params=pltpu.CompilerParams(
        use_tc_tiling_on_sc=False, needs_layout_passes=False
    ),
    out_type=jax.ShapeDtypeStruct((TOTAL_UNITS, EMB), jnp.float32),
    scratch_types=[
        pltpu.VMEM((CHUNK, N_CAT), jnp.int32),
        pltpu.VMEM((CHUNK, NUM_OUT), jnp.float32),
        pltpu.VMEM((NGATHER, 128), jnp.int32),
        pltpu.VMEM((NGATHER, 128), jnp.int32),
        pltpu.VMEM((UNITS, EMB), jnp.float32),
        pltpu.VMEM((UNITS, EMB), jnp.float32),
        pltpu.SemaphoreType.DMA,
        pltpu.SemaphoreType.DMA,
        pltpu.SemaphoreType.DMA,
        pltpu.SemaphoreType.DMA,
        pltpu.SemaphoreType.DMA,
    ],
)


@jax.jit
def kernel(num_tensor, cat_tensor, W, b, tables):
    num_emb = _num_matmul(num_tensor, W, b.reshape(1, NUM_OUT))
    tables_flat = tables.reshape(N_CAT * VOCAB, EMB)
    units = _sc_kernel(num_emb, cat_tensor, tables_flat)
    out = units.reshape(B // 8, TILES, 8, 4, EMB)
    return out.transpose(0, 2, 1, 3, 4).reshape(B, OUT_D)


# speedup vs baseline: 2.6720x; 2.6720x over previous
import functools

import jax
import jax.numpy as jnp
from jax import lax
from jax.experimental import pallas as pl
from jax.experimental.pallas import tpu as pltpu
from jax.experimental.pallas import tpu_sc as plsc

B = 16384
N_NUM = 13
NUM_OUT = 64
N_CAT = 26
VOCAB = 1000
EMB = 32
OUT_D = NUM_OUT + N_CAT * EMB

NC = 2
NS = 16
NW = NC * NS
NSPLIT = 2
BSPLIT = B // NSPLIT
ROWS_PER_W = BSPLIT // NW
CHUNK = 128
N_CHUNKS = ROWS_PER_W // CHUNK
LANES = 16


def _mm_body(x_ref, w_ref, b_ref, o_ref):
    o_ref[...] = (
        jnp.dot(x_ref[...], w_ref[...], preferred_element_type=jnp.float32)
        + b_ref[...]
    )


def _num_matmul(x, W, b2):
    MB = 2048
    return pl.pallas_call(
        _mm_body,
        grid=(B // MB,),
        in_specs=[
            pl.BlockSpec((MB, N_NUM), lambda i: (i, 0)),
            pl.BlockSpec((N_NUM, NUM_OUT), lambda i: (0, 0)),
            pl.BlockSpec((1, NUM_OUT), lambda i: (0, 0)),
        ],
        out_specs=pl.BlockSpec((MB, NUM_OUT), lambda i: (i, 0)),
        out_shape=jax.ShapeDtypeStruct((B, NUM_OUT), jnp.float32),
    )(x, W, b2)


def _sc_body(split, num_emb_hbm, catT_hbm, tables_hbm, out_hbm,
             idx_v, dest_v, num_v, gsem, osem, ssem):
    cid = lax.axis_index("c")
    sid = lax.axis_index("s")
    wid = sid * NC + cid
    row0 = split * BSPLIT + wid * ROWS_PER_W

    def chunk_body(ci, carry):
        base = pl.multiple_of(row0 + ci * CHUNK, CHUNK)
        obase = pl.multiple_of(base - split * BSPLIT, CHUNK)
        pltpu.sync_copy(catT_hbm.at[:, pl.ds(base, CHUNK)], idx_v)
        for f in range(N_CAT):
            off = f * VOCAB
            for j in range(CHUNK // LANES):
                sl = pl.ds(j * LANES, LANES)
                idx_v[f, sl] = idx_v[f, sl] + off
        gathers = [
            pltpu.async_copy(tables_hbm.at[idx_v.at[f]], dest_v.at[f], gsem)
            for f in range(N_CAT)
        ]
        pltpu.async_copy(num_emb_hbm.at[pl.ds(base, CHUNK)], num_v, ssem).wait()
        out_num = pltpu.async_copy(
            num_v, out_hbm.at[pl.ds(obase, CHUNK), pl.ds(0, NUM_OUT)], ssem
        )
        outs = []
        for f in range(N_CAT):
            gathers[f].wait()
            outs.append(
                pltpu.async_copy(
                    dest_v.at[f],
                    out_hbm.at[
                        pl.ds(obase, CHUNK), pl.ds(NUM_OUT + f * EMB, EMB)
                    ],
                    osem,
                )
            )
        out_num.wait()
        for o in outs:
            o.wait()
        return carry

    lax.fori_loop(0, N_CHUNKS, chunk_body, 0)


def _make_sc_kernel(split):
    return pl.kernel(
        functools.partial(_sc_body, split),
        mesh=plsc.VectorSubcoreMesh(core_axis_name="c", subcore_axis_name="s"),
        compiler_params=pltpu.CompilerParams(
            use_tc_tiling_on_sc=False, needs_layout_passes=False
        ),
        out_type=jax.ShapeDtypeStruct((BSPLIT, OUT_D), jnp.float32),
        scratch_types=[
            pltpu.VMEM((N_CAT, CHUNK), jnp.int32),
            pltpu.VMEM((N_CAT, CHUNK, EMB), jnp.float32),
            pltpu.VMEM((CHUNK, NUM_OUT), jnp.float32),
            pltpu.SemaphoreType.DMA,
            pltpu.SemaphoreType.DMA,
            pltpu.SemaphoreType.DMA,
        ],
    )


_sc_kernels = [_make_sc_kernel(s) for s in range(NSPLIT)]


@jax.jit
def kernel(num_tensor, cat_tensor, W, b, tables):
    num_emb = _num_matmul(num_tensor, W, b.reshape(1, NUM_OUT))
    catT = cat_tensor.T
    tables_flat = tables.reshape(N_CAT * VOCAB, EMB)
    pieces = [
        k(num_emb, catT, tables_flat) for k in _sc_kernels
    ]
    return jnp.concatenate(pieces, axis=0)
